# Initial kernel scaffold; baseline (speedup 1.0000x reference)
import jax
import jax.numpy as jnp
from jax.experimental import pallas as pl


def kernel(feat_trs, pred_trs, feat_tgt, pred_tgt, Centroid_trs, Amount_trs, Centroid_tgt, Amount_tgt):
    raise NotImplementedError("write your pallas kernel here")


# speedup vs baseline: 2.2026x; 2.2026x over previous
import functools

import jax
import jax.numpy as jnp
from jax import lax
from jax.experimental import pallas as pl
from jax.experimental.pallas import tpu as pltpu
from jax.experimental.pallas import tpu_sc as plsc

C = 19
CP = 20
FEAT = 256
N = 131072
NC = 2
NS = 16
NW = NC * NS
TOK_PER_W = N // NW
CHUNK = 128
NCHUNK = TOK_PER_W // CHUNK
LANES = 16
FVREG = FEAT // LANES


def _sc_body(f1, p1, f2, p2, os1, ocn1, os2, ocn2,
             fbuf, lbuf, acc1, cnt1, acc2, cnt2):
    wid = lax.axis_index("s") * NC + lax.axis_index("c")
    base = wid * TOK_PER_W
    ones = jnp.ones((LANES,), jnp.float32)
    zeros = jnp.zeros((LANES,), jnp.float32)

    def zero_row(r, _):
        for j in range(FVREG):
            acc1[r, pl.ds(j * LANES, LANES)] = zeros
            acc2[r, pl.ds(j * LANES, LANES)] = zeros
        cnt1[r, pl.ds(0, LANES)] = zeros
        cnt2[r, pl.ds(0, LANES)] = zeros
        return 0

    lax.fori_loop(0, CP, zero_row, 0)

    def do_stream(f_hbm, p_hbm, acc, cnt, osum, ocnt):
        pltpu.sync_copy(p_hbm.at[pl.ds(base, TOK_PER_W)], lbuf)

        def chunk_body(ci, _):
            pltpu.sync_copy(f_hbm.at[pl.ds(base + ci * CHUNK, CHUNK)], fbuf)

            def grp(g, _):
                lblv = lbuf[pl.ds(ci * CHUNK + g * LANES, LANES)]
                tbase = g * LANES
                for k in range(LANES):
                    lbl = lblv[k]
                    plsc.addupdate(cnt.at[lbl], ones)
                    for j in range(FVREG):
                        plsc.addupdate(acc.at[lbl, pl.ds(j * LANES, LANES)],
                                       fbuf[tbase + k, pl.ds(j * LANES, LANES)])
                return 0

            lax.fori_loop(0, CHUNK // LANES, grp, 0)
            return 0

        lax.fori_loop(0, NCHUNK, chunk_body, 0)
        pltpu.sync_copy(acc, osum.at[wid])
        pltpu.sync_copy(cnt, ocnt.at[wid])

    do_stream(f1, p1, acc1, cnt1, os1, ocn1)
    do_stream(f2, p2, acc2, cnt2, os2, ocn2)


_sc_partials = functools.partial(
    pl.kernel,
    mesh=plsc.VectorSubcoreMesh(core_axis_name="c", subcore_axis_name="s"),
    out_type=[
        jax.ShapeDtypeStruct((NW, CP, FEAT), jnp.float32),
        jax.ShapeDtypeStruct((NW, CP, LANES), jnp.float32),
        jax.ShapeDtypeStruct((NW, CP, FEAT), jnp.float32),
        jax.ShapeDtypeStruct((NW, CP, LANES), jnp.float32),
    ],
    scratch_types=[
        pltpu.VMEM((CHUNK, FEAT), jnp.float32),
        pltpu.VMEM((TOK_PER_W,), jnp.int32),
        pltpu.VMEM((CP, FEAT), jnp.float32),
        pltpu.VMEM((CP, LANES), jnp.float32),
        pltpu.VMEM((CP, FEAT), jnp.float32),
        pltpu.VMEM((CP, LANES), jnp.float32),
    ],
)(_sc_body)


def _epilogue(ps1, pc1, ps2, pc2, c1, a1, c2, a2, oc1, oa1, oc2, oa2):
    def one(ps, pc, cen, amt, oc, oa):
        sums = ps[0]
        cnts = pc[0]
        for i in range(1, NW):
            sums = sums + ps[i]
            cnts = cnts + pc[i]
        cnt19 = cnts[0:C, 0:1]
        sums19 = sums[0:C, :]
        amount_cxa = jnp.where(cnt19 == 0.0, 1.0, cnt19)
        mean = sums19 / amount_cxa
        denom = cnt19 + amt[...]
        safe = jnp.where(denom == 0.0, 1.0, denom)
        w = jnp.where(cnt19 == 0.0, 0.0, cnt19 / safe)
        oc[...] = cen[...] * (1.0 - w) + mean * w
        oa[...] = amt[...] + cnt19

    one(ps1, pc1, c1, a1, oc1, oa1)
    one(ps2, pc2, c2, a2, oc2, oa2)


def kernel(feat_trs, pred_trs, feat_tgt, pred_tgt,
           Centroid_trs, Amount_trs, Centroid_tgt, Amount_tgt):
    os1, ocn1, os2, ocn2 = _sc_partials(feat_trs, pred_trs, feat_tgt, pred_tgt)
    a1 = Amount_trs.reshape(C, 1)
    a2 = Amount_tgt.reshape(C, 1)
    oc1, oa1, oc2, oa2 = pl.pallas_call(
        _epilogue,
        out_shape=[
            jax.ShapeDtypeStruct((C, FEAT), jnp.float32),
            jax.ShapeDtypeStruct((C, 1), jnp.float32),
            jax.ShapeDtypeStruct((C, FEAT), jnp.float32),
            jax.ShapeDtypeStruct((C, 1), jnp.float32),
        ],
    )(os1, ocn1, os2, ocn2, Centroid_trs, a1, Centroid_tgt, a2)
    return (oc1, oa1.reshape(C), oc2, oa2.reshape(C))


# speedup vs baseline: 2.6446x; 1.2006x over previous
import functools

import jax
import jax.numpy as jnp
from jax import lax
from jax.experimental import pallas as pl
from jax.experimental.pallas import tpu as pltpu
from jax.experimental.pallas import tpu_sc as plsc

C = 19
CP = 20
FEAT = 256
N = 131072
NC = 2
NS = 16
NW = NC * NS
TOK_PER_W = N // NW
CHUNK = 128
NCHUNK = TOK_PER_W // CHUNK
LANES = 16
FVREG = FEAT // LANES
ACCW = CP * FEAT


def _sc_body(f1, p1, f2, p2, os1, ocn1, os2, ocn2,
             fbuf0, fbuf1, lbuf, acc1, cnt1, acc2, cnt2, sem0, sem1):
    wid = lax.axis_index("s") * NC + lax.axis_index("c")
    base = wid * TOK_PER_W
    ones = jnp.ones((LANES,), jnp.float32)
    zeros = jnp.zeros((LANES,), jnp.float32)

    def zero_body(r, _):
        o = r * LANES
        acc1[pl.ds(o, LANES)] = zeros
        cnt1[pl.ds(o, LANES)] = zeros
        acc2[pl.ds(o, LANES)] = zeros
        cnt2[pl.ds(o, LANES)] = zeros
        return 0

    lax.fori_loop(0, ACCW // LANES, zero_body, 0)

    def do_stream(f_hbm, p_hbm, acc, cnt, osum, ocnt):
        pltpu.sync_copy(p_hbm.at[pl.ds(base, TOK_PER_W)], lbuf)

        def start(ci, buf, sem):
            pltpu.async_copy(f_hbm.at[pl.ds(base + ci * CHUNK, CHUNK)], buf, sem)

        def wait(ci, buf, sem):
            pltpu.make_async_copy(
                f_hbm.at[pl.ds(base + ci * CHUNK, CHUNK)], buf, sem).wait()

        def process(buf, ci):
            def grp(g, _):
                lblv = lbuf[pl.ds(ci * CHUNK + g * LANES, LANES)]
                offv = lblv * FEAT
                tbase = g * LANES
                for k in range(LANES):
                    off = offv[k]
                    plsc.addupdate(cnt.at[pl.ds(off, LANES)], ones)
                    for j in range(FVREG):
                        plsc.addupdate(acc.at[pl.ds(off + j * LANES, LANES)],
                                       buf[tbase + k, pl.ds(j * LANES, LANES)])
                return 0

            lax.fori_loop(0, CHUNK // LANES, grp, 0)

        start(0, fbuf0, sem0)

        def body2(h, _):
            ci0 = 2 * h
            ci1 = 2 * h + 1
            start(ci1, fbuf1, sem1)
            wait(ci0, fbuf0, sem0)
            process(fbuf0, ci0)

            @pl.when(ci0 + 2 < NCHUNK)
            def _():
                start(ci0 + 2, fbuf0, sem0)

            wait(ci1, fbuf1, sem1)
            process(fbuf1, ci1)
            return 0

        lax.fori_loop(0, NCHUNK // 2, body2, 0)
        pltpu.sync_copy(acc, osum.at[wid])
        pltpu.sync_copy(cnt, ocnt.at[wid])

    do_stream(f1, p1, acc1, cnt1, os1, ocn1)
    do_stream(f2, p2, acc2, cnt2, os2, ocn2)


_sc_partials = functools.partial(
    pl.kernel,
    mesh=plsc.VectorSubcoreMesh(core_axis_name="c", subcore_axis_name="s"),
    out_type=[
        jax.ShapeDtypeStruct((NW, ACCW), jnp.float32),
        jax.ShapeDtypeStruct((NW, ACCW), jnp.float32),
        jax.ShapeDtypeStruct((NW, ACCW), jnp.float32),
        jax.ShapeDtypeStruct((NW, ACCW), jnp.float32),
    ],
    scratch_types=[
        pltpu.VMEM((CHUNK, FEAT), jnp.float32),
        pltpu.VMEM((CHUNK, FEAT), jnp.float32),
        pltpu.VMEM((TOK_PER_W,), jnp.int32),
        pltpu.VMEM((ACCW,), jnp.float32),
        pltpu.VMEM((ACCW,), jnp.float32),
        pltpu.VMEM((ACCW,), jnp.float32),
        pltpu.VMEM((ACCW,), jnp.float32),
        pltpu.SemaphoreType.DMA,
        pltpu.SemaphoreType.DMA,
    ],
)(_sc_body)


def _epilogue(ps1, pc1, ps2, pc2, c1, a1, c2, a2, oc1, oa1, oc2, oa2):
    def one(ps, pc, cen, amt, oc, oa):
        sums = ps[0]
        cnts = pc[0]
        for i in range(1, NW):
            sums = sums + ps[i]
            cnts = cnts + pc[i]
        cnt19 = cnts[0:C, 0:1]
        sums19 = sums[0:C, :]
        amount_cxa = jnp.where(cnt19 == 0.0, 1.0, cnt19)
        mean = sums19 / amount_cxa
        denom = cnt19 + amt[...]
        safe = jnp.where(denom == 0.0, 1.0, denom)
        w = jnp.where(cnt19 == 0.0, 0.0, cnt19 / safe)
        oc[...] = cen[...] * (1.0 - w) + mean * w
        oa[...] = amt[...] + cnt19

    one(ps1, pc1, c1, a1, oc1, oa1)
    one(ps2, pc2, c2, a2, oc2, oa2)


def kernel(feat_trs, pred_trs, feat_tgt, pred_tgt,
           Centroid_trs, Amount_trs, Centroid_tgt, Amount_tgt):
    os1, ocn1, os2, ocn2 = _sc_partials(feat_trs, pred_trs, feat_tgt, pred_tgt)
    os1 = os1.reshape(NW, CP, FEAT)
    ocn1 = ocn1.reshape(NW, CP, FEAT)
    os2 = os2.reshape(NW, CP, FEAT)
    ocn2 = ocn2.reshape(NW, CP, FEAT)
    a1 = Amount_trs.reshape(C, 1)
    a2 = Amount_tgt.reshape(C, 1)
    oc1, oa1, oc2, oa2 = pl.pallas_call(
        _epilogue,
        out_shape=[
            jax.ShapeDtypeStruct((C, FEAT), jnp.float32),
            jax.ShapeDtypeStruct((C, 1), jnp.float32),
            jax.ShapeDtypeStruct((C, FEAT), jnp.float32),
            jax.ShapeDtypeStruct((C, 1), jnp.float32),
        ],
    )(os1, ocn1, os2, ocn2, Centroid_trs, a1, Centroid_tgt, a2)
    return (oc1, oa1.reshape(C), oc2, oa2.reshape(C))


# speedup vs baseline: 5.2236x; 1.9752x over previous
import functools

import jax
import jax.numpy as jnp
from jax import lax
from jax.experimental import pallas as pl
from jax.experimental.pallas import tpu as pltpu
from jax.experimental.pallas import tpu_sc as plsc

C = 19
CP = 20
FEAT = 256
N = 131072
NC = 2
NS = 16
NW = NC * NS
TOK_PER_W = N // NW
CHUNK = 128
NCHUNK = TOK_PER_W // CHUNK
LANES = 16
FVREG = FEAT // LANES
ACCW = CP * FEAT


def _sc_body(f1, p1, f2, p2, os1, ocn1, os2, ocn2,
             fbuf0, fbuf1, lbuf, acc1, cnt1, acc2, cnt2, sem0, sem1):
    wid = lax.axis_index("s") * NC + lax.axis_index("c")
    base = wid * TOK_PER_W
    ones = jnp.ones((LANES,), jnp.float32)
    zeros = jnp.zeros((LANES,), jnp.float32)

    def zero_body(r, _):
        o = r * LANES
        acc1[pl.ds(o, LANES)] = zeros
        cnt1[pl.ds(o, LANES)] = zeros
        acc2[pl.ds(o, LANES)] = zeros
        cnt2[pl.ds(o, LANES)] = zeros
        return 0

    lax.fori_loop(0, ACCW // LANES, zero_body, 0)

    def do_stream(f_hbm, p_hbm, acc, cnt, osum, ocnt):
        pltpu.sync_copy(p_hbm.at[pl.ds(base, TOK_PER_W)], lbuf)

        def start(ci, buf, sem):
            pltpu.async_copy(f_hbm.at[pl.ds(base + ci * CHUNK, CHUNK)], buf, sem)

        def wait(ci, buf, sem):
            pltpu.make_async_copy(
                f_hbm.at[pl.ds(base + ci * CHUNK, CHUNK)], buf, sem).wait()

        def process(buf, ci):
            @plsc.parallel_loop(0, CHUNK // LANES, unroll=2)
            def grp(g):
                lblv = lbuf[pl.ds(ci * CHUNK + g * LANES, LANES)]
                offv = lblv * FEAT
                tbase = g * LANES

                def load_tok(k):
                    return [buf[tbase + k, pl.ds(j * LANES, LANES)]
                            for j in range(FVREG)]

                def store_tok(k, vals):
                    off = offv[k]
                    plsc.addupdate(cnt.at[pl.ds(off, LANES)], ones)
                    for j in range(FVREG):
                        plsc.addupdate(acc.at[pl.ds(off + j * LANES, LANES)],
                                       vals[j])

                vals = load_tok(0)
                for k in range(LANES):
                    nxt = load_tok(k + 1) if k + 1 < LANES else None
                    store_tok(k, vals)
                    vals = nxt

        start(0, fbuf0, sem0)

        def body2(h, _):
            ci0 = 2 * h
            ci1 = 2 * h + 1
            start(ci1, fbuf1, sem1)
            wait(ci0, fbuf0, sem0)
            process(fbuf0, ci0)

            @pl.when(ci0 + 2 < NCHUNK)
            def _():
                start(ci0 + 2, fbuf0, sem0)

            wait(ci1, fbuf1, sem1)
            process(fbuf1, ci1)
            return 0

        lax.fori_loop(0, NCHUNK // 2, body2, 0)
        pltpu.sync_copy(acc, osum.at[wid])
        pltpu.sync_copy(cnt, ocnt.at[wid])

    do_stream(f1, p1, acc1, cnt1, os1, ocn1)
    do_stream(f2, p2, acc2, cnt2, os2, ocn2)


_sc_partials = functools.partial(
    pl.kernel,
    mesh=plsc.VectorSubcoreMesh(core_axis_name="c", subcore_axis_name="s"),
    out_type=[
        jax.ShapeDtypeStruct((NW, ACCW), jnp.float32),
        jax.ShapeDtypeStruct((NW, ACCW), jnp.float32),
        jax.ShapeDtypeStruct((NW, ACCW), jnp.float32),
        jax.ShapeDtypeStruct((NW, ACCW), jnp.float32),
    ],
    scratch_types=[
        pltpu.VMEM((CHUNK, FEAT), jnp.float32),
        pltpu.VMEM((CHUNK, FEAT), jnp.float32),
        pltpu.VMEM((TOK_PER_W,), jnp.int32),
        pltpu.VMEM((ACCW,), jnp.float32),
        pltpu.VMEM((ACCW,), jnp.float32),
        pltpu.VMEM((ACCW,), jnp.float32),
        pltpu.VMEM((ACCW,), jnp.float32),
        pltpu.SemaphoreType.DMA,
        pltpu.SemaphoreType.DMA,
    ],
)(_sc_body)


def _epilogue(ps1, pc1, ps2, pc2, c1, a1, c2, a2, oc1, oa1, oc2, oa2):
    def one(ps, pc, cen, amt, oc, oa):
        sums = ps[0]
        cnts = pc[0]
        for i in range(1, NW):
            sums = sums + ps[i]
            cnts = cnts + pc[i]
        cnt19 = cnts[0:C, 0:1]
        sums19 = sums[0:C, :]
        amount_cxa = jnp.where(cnt19 == 0.0, 1.0, cnt19)
        mean = sums19 / amount_cxa
        denom = cnt19 + amt[...]
        safe = jnp.where(denom == 0.0, 1.0, denom)
        w = jnp.where(cnt19 == 0.0, 0.0, cnt19 / safe)
        oc[...] = cen[...] * (1.0 - w) + mean * w
        oa[...] = amt[...] + cnt19

    one(ps1, pc1, c1, a1, oc1, oa1)
    one(ps2, pc2, c2, a2, oc2, oa2)


def kernel(feat_trs, pred_trs, feat_tgt, pred_tgt,
           Centroid_trs, Amount_trs, Centroid_tgt, Amount_tgt):
    os1, ocn1, os2, ocn2 = _sc_partials(feat_trs, pred_trs, feat_tgt, pred_tgt)
    os1 = os1.reshape(NW, CP, FEAT)
    ocn1 = ocn1.reshape(NW, CP, FEAT)
    os2 = os2.reshape(NW, CP, FEAT)
    ocn2 = ocn2.reshape(NW, CP, FEAT)
    a1 = Amount_trs.reshape(C, 1)
    a2 = Amount_tgt.reshape(C, 1)
    oc1, oa1, oc2, oa2 = pl.pallas_call(
        _epilogue,
        out_shape=[
            jax.ShapeDtypeStruct((C, FEAT), jnp.float32),
            jax.ShapeDtypeStruct((C, 1), jnp.float32),
            jax.ShapeDtypeStruct((C, FEAT), jnp.float32),
            jax.ShapeDtypeStruct((C, 1), jnp.float32),
        ],
    )(os1, ocn1, os2, ocn2, Centroid_trs, a1, Centroid_tgt, a2)
    return (oc1, oa1.reshape(C), oc2, oa2.reshape(C))


# speedup vs baseline: 11.3438x; 2.1717x over previous
import functools

import jax
import jax.numpy as jnp
from jax import lax
from jax.experimental import pallas as pl
from jax.experimental.pallas import tpu as pltpu
from jax.experimental.pallas import tpu_sc as plsc

C = 19
CP = 20
CROWS = 32
FEAT = 256
N = 131072
NC = 2
NS = 16
NW = NC * NS
TOK_PER_W = N // NW
CHUNK = 128
NCHUNK = TOK_PER_W // CHUNK
LANES = 16
FVREG = FEAT // LANES
ACCW = CP * FEAT
BT = 2048
NB = N // BT


def _sc_body(f_hbm, p_hbm, osum, ocnt,
             fbuf0, fbuf1, lbuf, acc, cnt, sem0, sem1):
    wid = lax.axis_index("s") * NC + lax.axis_index("c")
    base = wid * TOK_PER_W
    ones = jnp.ones((LANES,), jnp.float32)
    zeros = jnp.zeros((LANES,), jnp.float32)

    def zero_body(r, _):
        o = r * LANES
        acc[pl.ds(o, LANES)] = zeros
        cnt[pl.ds(o, LANES)] = zeros
        return 0

    lax.fori_loop(0, ACCW // LANES, zero_body, 0)

    pltpu.sync_copy(p_hbm.at[pl.ds(base, TOK_PER_W)],
                    lbuf.at[pl.ds(0, TOK_PER_W)])

    def start(ci, buf, sem):
        pltpu.async_copy(f_hbm.at[pl.ds(base + ci * CHUNK, CHUNK)], buf, sem)

    def wait(ci, buf, sem):
        pltpu.make_async_copy(
            f_hbm.at[pl.ds(base + ci * CHUNK, CHUNK)], buf, sem).wait()

    def process(buf, ci):
        @plsc.parallel_loop(0, CHUNK, unroll=4)
        def tok(t):
            lsp = lbuf[pl.ds(ci * CHUNK + t, LANES)]
            off = (lsp * FEAT)[0]
            vals = [buf[t, pl.ds(j * LANES, LANES)] for j in range(FVREG)]
            plsc.addupdate(cnt.at[pl.ds(off, LANES)], ones)
            for j in range(FVREG):
                plsc.addupdate(acc.at[pl.ds(off + j * LANES, LANES)],
                               vals[j])

    start(0, fbuf0, sem0)

    def body2(h, _):
        ci0 = 2 * h
        ci1 = 2 * h + 1
        start(ci1, fbuf1, sem1)
        wait(ci0, fbuf0, sem0)
        process(fbuf0, ci0)

        @pl.when(ci0 + 2 < NCHUNK)
        def _():
            start(ci0 + 2, fbuf0, sem0)

        wait(ci1, fbuf1, sem1)
        process(fbuf1, ci1)
        return 0

    lax.fori_loop(0, NCHUNK // 2, body2, 0)
    pltpu.sync_copy(acc, osum.at[wid])
    pltpu.sync_copy(cnt, ocnt.at[wid])


_sc_partials = functools.partial(
    pl.kernel,
    mesh=plsc.VectorSubcoreMesh(core_axis_name="c", subcore_axis_name="s"),
    out_type=[
        jax.ShapeDtypeStruct((NW, ACCW), jnp.float32),
        jax.ShapeDtypeStruct((NW, ACCW), jnp.float32),
    ],
    scratch_types=[
        pltpu.VMEM((CHUNK, FEAT), jnp.float32),
        pltpu.VMEM((CHUNK, FEAT), jnp.float32),
        pltpu.VMEM((TOK_PER_W + LANES,), jnp.int32),
        pltpu.VMEM((ACCW,), jnp.float32),
        pltpu.VMEM((ACCW,), jnp.float32),
        pltpu.SemaphoreType.DMA,
        pltpu.SemaphoreType.DMA,
    ],
)(_sc_body)


def _centroid_update(cnt19, sums19, cen, amt):
    amount_cxa = jnp.where(cnt19 == 0.0, 1.0, cnt19)
    mean = sums19 / amount_cxa
    denom = cnt19 + amt
    safe = jnp.where(denom == 0.0, 1.0, denom)
    w = jnp.where(cnt19 == 0.0, 0.0, cnt19 / safe)
    return cen * (1.0 - w) + mean * w, amt + cnt19


def _tc_onehot_body(lab_ref, feat_ref, cen_ref, amt_ref, oc_ref, oa_ref,
                    acc, cnta):
    i = pl.program_id(0)

    @pl.when(i == 0)
    def _():
        acc[...] = jnp.zeros((CROWS, FEAT), jnp.float32)
        cnta[...] = jnp.zeros((CROWS, 128), jnp.float32)

    lbl = lab_ref[0, 0, :]
    oh = (lax.broadcasted_iota(jnp.int32, (CROWS, BT), 0)
          == jnp.broadcast_to(lbl[None, :], (CROWS, BT))).astype(jnp.float32)
    acc[...] += jax.lax.dot_general(
        oh, feat_ref[...], (((1,), (0,)), ((), ())),
        preferred_element_type=jnp.float32)
    cnta[...] += jnp.broadcast_to(
        jnp.sum(oh, axis=1, keepdims=True), (CROWS, 128))

    @pl.when(i == NB - 1)
    def _():
        cnt19 = cnta[0:C, 0:1]
        sums19 = acc[0:C, :]
        oc, oa = _centroid_update(cnt19, sums19, cen_ref[...], amt_ref[...])
        oc_ref[...] = oc
        oa_ref[...] = oa


def _tc_onehot(labels3d, feat, cen, amt):
    return pl.pallas_call(
        _tc_onehot_body,
        grid=(NB,),
        in_specs=[
            pl.BlockSpec((1, 1, BT), lambda i: (i, 0, 0)),
            pl.BlockSpec((BT, FEAT), lambda i: (i, 0)),
            pl.BlockSpec((C, FEAT), lambda i: (0, 0)),
            pl.BlockSpec((C, 1), lambda i: (0, 0)),
        ],
        out_specs=[
            pl.BlockSpec((C, FEAT), lambda i: (0, 0)),
            pl.BlockSpec((C, 1), lambda i: (0, 0)),
        ],
        out_shape=[
            jax.ShapeDtypeStruct((C, FEAT), jnp.float32),
            jax.ShapeDtypeStruct((C, 1), jnp.float32),
        ],
        scratch_shapes=[
            pltpu.VMEM((CROWS, FEAT), jnp.float32),
            pltpu.VMEM((CROWS, 128), jnp.float32),
        ],
    )(labels3d, feat, cen, amt)


def _epilogue(ps, pc, cen, amt, oc_ref, oa_ref):
    sums = ps[0]
    cnts = pc[0]
    for i in range(1, NW):
        sums = sums + ps[i]
        cnts = cnts + pc[i]
    cnt19 = cnts[0:C, 0:1]
    sums19 = sums[0:C, :]
    oc, oa = _centroid_update(cnt19, sums19, cen[...], amt[...])
    oc_ref[...] = oc
    oa_ref[...] = oa


def kernel(feat_trs, pred_trs, feat_tgt, pred_tgt,
           Centroid_trs, Amount_trs, Centroid_tgt, Amount_tgt):
    os2, ocn2 = _sc_partials(feat_tgt, pred_tgt)
    a1 = Amount_trs.reshape(C, 1)
    oc1, oa1 = _tc_onehot(pred_trs.reshape(NB, 1, BT), feat_trs,
                          Centroid_trs, a1)
    a2 = Amount_tgt.reshape(C, 1)
    oc2, oa2 = pl.pallas_call(
        _epilogue,
        out_shape=[
            jax.ShapeDtypeStruct((C, FEAT), jnp.float32),
            jax.ShapeDtypeStruct((C, 1), jnp.float32),
        ],
    )(os2.reshape(NW, CP, FEAT), ocn2.reshape(NW, CP, FEAT),
      Centroid_tgt, a2)
    return (oc1, oa1.reshape(C), oc2, oa2.reshape(C))
